# baseline (speedup 1.0000x reference)
import jax
import jax.numpy as jnp
from jax import lax
from jax.experimental import pallas as pl
from jax.experimental.pallas import tpu as pltpu
from jax.experimental.pallas import tpu_sc as plsc

VOCAB = 1_000_000
D = 64
B = 4096
CTX = 20
CTXP = 24
NCORES = 2
NSUB = 16
NW = NCORES * NSUB
NIDX = B * CTX
IDXCH = 2048
HCAP = 16384
NROUNDS = (NIDX + HCAP - 1) // HCAP
SW = 512
NSTRIP = (VOCAB + SW - 1) // SW
TAIL0 = (NSTRIP - 1) * SW
TAILW = VOCAB - TAIL0
VB = 32768


def _sc_gather_sum(idx_hbm, tabt_hbm, tail_hbm, out_hbm,
                   idxb_v, hit_v, strip_v, acc_v, sem):
    c = lax.axis_index("c")
    s_ax = lax.axis_index("s")
    wid = s_ax * NCORES + c
    i32 = jnp.int32
    iota16 = lax.iota(i32, 16)
    w16 = jnp.full((16,), wid, i32)
    zero = jnp.zeros((16,), jnp.float32)
    for l in range(CTXP):
        for j in range(128 // 16):
            acc_v[l, pl.ds(j * 16, 16)] = zero

    def scan_round(lo):
        def chunk_body(ch, gn):
            pltpu.sync_copy(idx_hbm.at[pl.ds(ch * IDXCH, IDXCH)], idxb_v)

            def vec_body(t, gn):
                iv = idxb_v[pl.ds(t * 16, 16)]
                mine = ((iv >> 9) & 31) == w16
                gpos = ch * IDXCH + t * 16
                l16 = (gpos + iota16) % CTX
                packed = (iv << 5) | l16
                m = mine.astype(i32)
                incl = plsc.cumsum(m)
                excl = incl - m
                seq = jnp.full((16,), gn, i32) + excl
                acc_m = mine & (seq >= lo) & (seq < lo + HCAP)
                pos = (seq - lo) & (HCAP - 1)
                plsc.store_scatter(hit_v, [pos], packed, mask=acc_m)
                return gn + jnp.sum(m)

            return lax.fori_loop(0, IDXCH // 16, vec_body, gn)

        return lax.fori_loop(0, NIDX // IDXCH, chunk_body, i32(0))

    def drain(nr):
        nk = jnp.where(wid < NSTRIP - (NSTRIP // NW) * NW, NSTRIP // NW + 1,
                       NSTRIP // NW)
        nv = (nr + 15) // 16

        def strip_body(k, _):
            s = wid + NW * k
            s16 = jnp.full((16,), s, i32)

            @pl.when(s < NSTRIP - 1)
            def _stage_full():
                hs = []
                for tr in range(8):
                    for tc in range(4):
                        hs.append(pltpu.async_copy(
                            tabt_hbm.at[pl.ds(tr * 8, 8),
                                        pl.ds(s * SW + tc * 128, 128)],
                            strip_v.at[tr, tc], sem))
                for h in hs:
                    h.wait()

            @pl.when(s == NSTRIP - 1)
            def _stage_tail():
                hs = []
                for tr in range(8):
                    hs.append(pltpu.async_copy(
                        tail_hbm.at[pl.ds(tr * 8, 8), pl.ds(0, 128)],
                        strip_v.at[tr, 0], sem))
                for h in hs:
                    h.wait()

            def hit_body(h, _):
                ph = hit_v[pl.ds(h * 16, 16)]
                vv = lax.shift_right_logical(ph, 5)
                ll = jnp.minimum(ph & 31, CTX - 1)
                valid = (h * 16 + iota16) < jnp.full((16,), nr, i32)
                mine = valid & ((vv >> 9) == s16)
                nhit = jnp.sum(mine.astype(i32))

                @pl.when(nhit > 0)
                def _process():
                    u = (vv - s * SW) & (SW - 1)
                    tc16 = lax.shift_right_logical(u, 7)
                    ln16 = u & 127
                    for tr in range(8):
                        tr16 = jnp.full((16,), tr, i32)
                        for sub in range(8):
                            sub16 = jnp.full((16,), sub, i32)
                            d16 = jnp.full((16,), tr * 8 + sub, i32)
                            vals = plsc.load_gather(
                                strip_v, [tr16, tc16, sub16, ln16], mask=mine)
                            plsc.addupdate_scatter(
                                acc_v, [ll, d16], vals, mask=mine)

                return _

            lax.fori_loop(0, nv, hit_body, 0)
            return _

        lax.fori_loop(0, nk, strip_body, 0)

    gnt = scan_round(0)
    drain(jnp.minimum(gnt, HCAP))
    for r in range(1, NROUNDS):
        @pl.when(gnt > r * HCAP)
        def _extra_round():
            scan_round(r * HCAP)
            drain(jnp.minimum(gnt - r * HCAP, HCAP))

    pltpu.sync_copy(acc_v, out_hbm.at[wid])


def _sc_partial_sums(idx_flat, tabt, tail_pad):
    mesh = plsc.VectorSubcoreMesh(core_axis_name="c", subcore_axis_name="s")
    return pl.kernel(
        _sc_gather_sum,
        out_type=jax.ShapeDtypeStruct((NW, CTXP, 128), jnp.float32),
        mesh=mesh,
        scratch_types=[
            pltpu.VMEM((IDXCH,), jnp.int32),
            pltpu.VMEM((HCAP,), jnp.int32),
            pltpu.VMEM((8, 4, 8, 128), jnp.float32),
            pltpu.VMEM((CTXP, 128), jnp.float32),
            pltpu.SemaphoreType.DMA,
        ],
        compiler_params=pltpu.CompilerParams(
            use_tc_tiling_on_sc=True, needs_layout_passes=False
        ),
    )(idx_flat, tabt, tail_pad)


def _mm_body(part_ref, wt_ref, b_ref, out_ref):
    combined = jnp.sum(part_ref[...], axis=0)[:CTX, :D] * (1.0 / B)
    out_ref[...] = (
        lax.dot_general(
            combined,
            wt_ref[...],
            (((1,), (0,)), ((), ())),
            preferred_element_type=jnp.float32,
        )
        + b_ref[...]
    )


def _tc_matmul(partials, Wt, b2d):
    return pl.pallas_call(
        _mm_body,
        grid=(pl.cdiv(VOCAB, VB),),
        in_specs=[
            pl.BlockSpec((NW, CTXP, 128), lambda i: (0, 0, 0)),
            pl.BlockSpec((D, VB), lambda i: (0, i)),
            pl.BlockSpec((1, VB), lambda i: (0, i)),
        ],
        out_specs=pl.BlockSpec((CTX, VB), lambda i: (0, i)),
        out_shape=jax.ShapeDtypeStruct((CTX, VOCAB), jnp.float32),
    )(partials, Wt, b2d)


def kernel(context_idxs, emb_table, W, b):
    idx_flat = context_idxs.reshape(-1).astype(jnp.int32)
    tabt = emb_table.T
    tail_pad = jnp.pad(tabt[:, TAIL0:], ((0, 0), (0, 128 - TAILW)))
    partials = _sc_partial_sums(idx_flat, tabt, tail_pad)
    return _tc_matmul(partials, W.T, b.reshape(1, VOCAB))
